# Initial kernel scaffold; baseline (speedup 1.0000x reference)
import jax
import jax.numpy as jnp
from jax.experimental import pallas as pl


def kernel(x, edge_index, W_in, b_in, W1_0, b1_0, W2_0, b2_0, W1_1, b1_1, W2_1, b2_1, ln_g, ln_b, W_out, b_out):
    raise NotImplementedError("write your pallas kernel here")


# speedup vs baseline: 10.5766x; 10.5766x over previous
import functools

import jax
import jax.numpy as jnp
from jax import lax
from jax.experimental import pallas as pl
from jax.experimental.pallas import tpu as pltpu
from jax.experimental.pallas import tpu_sc as plsc

N = 10000
D = 128
NP = 10240
E = 320000
NC = 2
NS = 16
NW = NC * NS
CH = 128
NCHUNK = 79
EPT = NCHUNK * CH
EPAD = NW * EPT
RPT = NP // NS
BR = 256
EPS = 1e-5

_mesh = plsc.VectorSubcoreMesh(core_axis_name="c", subcore_axis_name="s")


def _deg_body(dst_hbm, out_hbm, dstv, degv):
    c = lax.axis_index("c")
    s = lax.axis_index("s")
    wid = s * NC + c
    pltpu.sync_copy(dst_hbm.at[wid], dstv)
    zeros16 = jnp.zeros((16,), jnp.float32)

    def zbody(i, carry):
        degv[pl.ds(i * 16, 16)] = zeros16
        return carry

    lax.fori_loop(0, NP // 16, zbody, 0)
    ones16 = jnp.ones((16,), jnp.float32)

    def ebody(j, carry):
        for k in range(CH // 16):
            idx = dstv[j, pl.ds(k * 16, 16)]
            plsc.addupdate_scatter(degv, [idx], ones16)
        return carry

    lax.fori_loop(0, NCHUNK, ebody, 0)
    pltpu.sync_copy(degv, out_hbm.at[wid])


_sc_params = pltpu.CompilerParams(needs_layout_passes=False)

_deg_call = functools.partial(
    pl.kernel,
    out_type=jax.ShapeDtypeStruct((NW, NP), jnp.float32),
    mesh=_mesh,
    compiler_params=_sc_params,
    scratch_types=[
        pltpu.VMEM((NCHUNK, CH), jnp.int32),
        pltpu.VMEM((NP,), jnp.float32),
    ],
)(_deg_body)


def _agg_body(hs_hbm, src_hbm, dst_hbm, out_hbm, srcv, dstv, rows, acc_sh, gsem):
    c = lax.axis_index("c")
    s = lax.axis_index("s")
    wid = s * NC + c
    pltpu.sync_copy(src_hbm.at[wid], srcv)
    pltpu.sync_copy(dst_hbm.at[wid], dstv)
    zeros16 = jnp.zeros((16,), jnp.float32)

    def zbody(i, carry):
        for k in range(D // 16):
            rows[i, pl.ds(k * 16, 16)] = zeros16
        return carry

    lax.fori_loop(0, CH, zbody, 0)
    for q in range(RPT // CH):
        pltpu.sync_copy(rows, acc_sh.at[pl.ds(s * RPT + q * CH, CH)])
    plsc.subcore_barrier()

    def cbody(j, carry):
        pltpu.async_copy(hs_hbm.at[srcv.at[j]], rows, gsem).wait()
        pltpu.sync_copy(rows, acc_sh.at[dstv.at[j]], add=True)
        return carry

    lax.fori_loop(0, NCHUNK, cbody, 0)
    plsc.subcore_barrier()
    pltpu.sync_copy(acc_sh.at[pl.ds(s * RPT, RPT)],
                    out_hbm.at[c, pl.ds(s * RPT, RPT)])


_agg_call = functools.partial(
    pl.kernel,
    out_type=jax.ShapeDtypeStruct((NC, NP, D), jnp.float32),
    mesh=_mesh,
    compiler_params=_sc_params,
    scratch_types=[
        pltpu.VMEM((NCHUNK, CH), jnp.int32),
        pltpu.VMEM((NCHUNK, CH), jnp.int32),
        pltpu.VMEM((CH, D), jnp.float32),
        pltpu.VMEM_SHARED((NP, D), jnp.float32),
        pltpu.SemaphoreType.DMA,
    ],
)(_agg_body)


def _dinv_of(degp):
    return lax.rsqrt(jnp.maximum(jnp.sum(degp, axis=0), 1.0))


def _rowmask(i):
    rid = lax.broadcasted_iota(jnp.int32, (BR, 1), 0) + i * BR
    return rid < N


def _tc_in_body(x_ref, w_ref, b_ref, degp_ref, o_ref):
    i = pl.program_id(0)
    dinv = _dinv_of(degp_ref[...])
    h = jnp.dot(x_ref[...], w_ref[...], preferred_element_type=jnp.float32)
    h = jnp.maximum(h + b_ref[...], 0.0)
    o_ref[...] = jnp.where(_rowmask(i), h * dinv[:, None], 0.0)


def _tc_layer_body(p_ref, degp_ref, w1_ref, b1_ref, w2_ref, b2_ref, o_ref):
    i = pl.program_id(0)
    dinv = _dinv_of(degp_ref[...])
    t = (p_ref[0] + p_ref[1]) * dinv[:, None]
    z = jnp.dot(t, w1_ref[...], preferred_element_type=jnp.float32)
    z = jnp.maximum(z + b1_ref[...], 0.0)
    h = jnp.dot(z, w2_ref[...], preferred_element_type=jnp.float32) + b2_ref[...]
    o_ref[...] = jnp.where(_rowmask(i), h * dinv[:, None], 0.0)


def _tc_final_body(p_ref, degp_ref, w1_ref, b1_ref, w2_ref, b2_ref,
                   g_ref, bb_ref, wo_ref, bo_ref, o_ref):
    dinv = _dinv_of(degp_ref[...])
    t = (p_ref[0] + p_ref[1]) * dinv[:, None]
    z = jnp.dot(t, w1_ref[...], preferred_element_type=jnp.float32)
    z = jnp.maximum(z + b1_ref[...], 0.0)
    h = jnp.dot(z, w2_ref[...], preferred_element_type=jnp.float32) + b2_ref[...]
    mu = jnp.mean(h, axis=-1, keepdims=True)
    var = jnp.mean((h - mu) ** 2, axis=-1, keepdims=True)
    hn = (h - mu) * lax.rsqrt(var + EPS) * g_ref[...] + bb_ref[...]
    o_ref[...] = jnp.dot(hn, wo_ref[...], preferred_element_type=jnp.float32) + bo_ref[...]


def _vec_spec():
    return pl.BlockSpec((1, D), lambda i: (0, 0))


def _mat_spec():
    return pl.BlockSpec((D, D), lambda i: (0, 0))


def _row_spec():
    return pl.BlockSpec((BR, D), lambda i: (i, 0))


def _degp_spec():
    return pl.BlockSpec((NW, BR), lambda i: (0, i))


def _part_spec():
    return pl.BlockSpec((NC, BR, D), lambda i: (0, i, 0))


_GRID = NP // BR

_tc_in = pl.pallas_call(
    _tc_in_body,
    grid=(_GRID,),
    in_specs=[_row_spec(), _mat_spec(), _vec_spec(), _degp_spec()],
    out_specs=_row_spec(),
    out_shape=jax.ShapeDtypeStruct((NP, D), jnp.float32),
)

_tc_layer = pl.pallas_call(
    _tc_layer_body,
    grid=(_GRID,),
    in_specs=[_part_spec(), _degp_spec(), _mat_spec(), _vec_spec(),
              _mat_spec(), _vec_spec()],
    out_specs=_row_spec(),
    out_shape=jax.ShapeDtypeStruct((NP, D), jnp.float32),
)

_tc_final = pl.pallas_call(
    _tc_final_body,
    grid=(_GRID,),
    in_specs=[_part_spec(), _degp_spec(), _mat_spec(), _vec_spec(),
              _mat_spec(), _vec_spec(), _vec_spec(), _vec_spec(),
              _mat_spec(), _vec_spec()],
    out_specs=_row_spec(),
    out_shape=jax.ShapeDtypeStruct((NP, D), jnp.float32),
)


def kernel(x, edge_index, W_in, b_in, W1_0, b1_0, W2_0, b2_0,
           W1_1, b1_1, W2_1, b2_1, ln_g, ln_b, W_out, b_out):
    src = edge_index[0]
    dst = edge_index[1]
    pad = jnp.full((EPAD - E,), N, jnp.int32)
    src_p = jnp.concatenate([src, pad]).reshape(NW, NCHUNK, CH)
    dst_p = jnp.concatenate([dst, pad]).reshape(NW, NCHUNK, CH)
    x_p = jnp.pad(x, ((0, NP - N), (0, 0)))

    degp = _deg_call(dst_p)
    hs0 = _tc_in(x_p, W_in, b_in.reshape(1, D), degp)
    p0 = _agg_call(hs0, src_p, dst_p)
    hs1 = _tc_layer(p0, degp, W1_0, b1_0.reshape(1, D), W2_0, b2_0.reshape(1, D))
    p1 = _agg_call(hs1, src_p, dst_p)
    out = _tc_final(p1, degp, W1_1, b1_1.reshape(1, D), W2_1, b2_1.reshape(1, D),
                    ln_g.reshape(1, D), ln_b.reshape(1, D), W_out, b_out.reshape(1, D))
    return out[:N]


# speedup vs baseline: 11.8649x; 1.1218x over previous
import functools

import jax
import jax.numpy as jnp
from jax import lax
from jax.experimental import pallas as pl
from jax.experimental.pallas import tpu as pltpu
from jax.experimental.pallas import tpu_sc as plsc

N = 10000
D = 128
NP = 10240
E = 320000
NC = 2
NS = 16
NW = NC * NS
CH = 128
NCHUNK = 79
EPT = NCHUNK * CH
EPAD = NW * EPT
RPT = NP // NS
BR = 256
EPS = 1e-5

_mesh = plsc.VectorSubcoreMesh(core_axis_name="c", subcore_axis_name="s")


def _deg_body(dst_hbm, out_hbm, dstv, degv):
    c = lax.axis_index("c")
    s = lax.axis_index("s")
    wid = s * NC + c
    pltpu.sync_copy(dst_hbm.at[wid], dstv)
    zeros16 = jnp.zeros((16,), jnp.float32)

    def zbody(i, carry):
        degv[pl.ds(i * 16, 16)] = zeros16
        return carry

    lax.fori_loop(0, NP // 16, zbody, 0)
    ones16 = jnp.ones((16,), jnp.float32)

    def ebody(j, carry):
        for k in range(CH // 16):
            idx = dstv[j, pl.ds(k * 16, 16)]
            plsc.addupdate_scatter(degv, [idx], ones16)
        return carry

    lax.fori_loop(0, NCHUNK, ebody, 0)
    pltpu.sync_copy(degv, out_hbm.at[wid])


_sc_params = pltpu.CompilerParams(needs_layout_passes=False)

_deg_call = functools.partial(
    pl.kernel,
    out_type=jax.ShapeDtypeStruct((NW, NP), jnp.float32),
    mesh=_mesh,
    compiler_params=_sc_params,
    scratch_types=[
        pltpu.VMEM((NCHUNK, CH), jnp.int32),
        pltpu.VMEM((NP,), jnp.float32),
    ],
)(_deg_body)


def _agg_body(hs_hbm, src_hbm, dst_hbm, out_hbm, sidx, didx, rows, acc_sh,
              isem, gsem):
    c = lax.axis_index("c")
    s = lax.axis_index("s")
    wid = s * NC + c
    zeros16 = jnp.zeros((16,), jnp.float32)

    def zbody(i, carry):
        for k in range(D // 16):
            rows[0, i, pl.ds(k * 16, 16)] = zeros16
        return carry

    lax.fori_loop(0, CH, zbody, 0)
    for q in range(RPT // CH):
        pltpu.sync_copy(rows.at[0], acc_sh.at[pl.ds(s * RPT + q * CH, CH)])
    plsc.subcore_barrier()

    for p in range(3):
        pltpu.async_copy(src_hbm.at[wid, p], sidx.at[p], isem)
        pltpu.async_copy(dst_hbm.at[wid, p], didx.at[p], isem)
    pltpu.make_async_copy(src_hbm.at[wid, 0], sidx.at[0], isem).wait()
    pltpu.make_async_copy(dst_hbm.at[wid, 0], didx.at[0], isem).wait()
    pltpu.async_copy(hs_hbm.at[sidx.at[0]], rows.at[0], gsem)

    def cbody(j, carry):
        b = lax.rem(j, 2)
        pltpu.make_async_copy(hs_hbm.at[sidx.at[lax.rem(j, 4)]],
                              rows.at[b], gsem).wait()

        @pl.when(j + 1 < NCHUNK)
        def _():
            nxt = lax.rem(j + 1, 4)
            pltpu.make_async_copy(src_hbm.at[wid, j + 1], sidx.at[nxt],
                                  isem).wait()
            pltpu.make_async_copy(dst_hbm.at[wid, j + 1], didx.at[nxt],
                                  isem).wait()
            pltpu.async_copy(hs_hbm.at[sidx.at[nxt]], rows.at[1 - b], gsem)

        @pl.when(j + 3 < NCHUNK)
        def _():
            pltpu.async_copy(src_hbm.at[wid, j + 3], sidx.at[lax.rem(j + 3, 4)],
                             isem)
            pltpu.async_copy(dst_hbm.at[wid, j + 3], didx.at[lax.rem(j + 3, 4)],
                             isem)

        pltpu.sync_copy(rows.at[b], acc_sh.at[didx.at[lax.rem(j, 4)]], add=True)
        return carry

    lax.fori_loop(0, NCHUNK, cbody, 0)
    plsc.subcore_barrier()
    pltpu.sync_copy(acc_sh.at[pl.ds(s * RPT, RPT)],
                    out_hbm.at[c, pl.ds(s * RPT, RPT)])


_agg_call = functools.partial(
    pl.kernel,
    out_type=jax.ShapeDtypeStruct((NC, NP, D), jnp.float32),
    mesh=_mesh,
    compiler_params=_sc_params,
    scratch_types=[
        pltpu.VMEM((4, CH), jnp.int32),
        pltpu.VMEM((4, CH), jnp.int32),
        pltpu.VMEM((2, CH, D), jnp.float32),
        pltpu.VMEM_SHARED((NP, D), jnp.float32),
        pltpu.SemaphoreType.DMA,
        pltpu.SemaphoreType.DMA,
    ],
)(_agg_body)


def _dinv_of(degp):
    return lax.rsqrt(jnp.maximum(jnp.sum(degp, axis=0), 1.0))


def _rowmask(i):
    rid = lax.broadcasted_iota(jnp.int32, (BR, 1), 0) + i * BR
    return rid < N


def _tc_in_body(x_ref, w_ref, b_ref, degp_ref, o_ref):
    i = pl.program_id(0)
    dinv = _dinv_of(degp_ref[...])
    h = jnp.dot(x_ref[...], w_ref[...], preferred_element_type=jnp.float32)
    h = jnp.maximum(h + b_ref[...], 0.0)
    o_ref[...] = jnp.where(_rowmask(i), h * dinv[:, None], 0.0)


def _tc_layer_body(p_ref, degp_ref, w1_ref, b1_ref, w2_ref, b2_ref, o_ref):
    i = pl.program_id(0)
    dinv = _dinv_of(degp_ref[...])
    t = (p_ref[0] + p_ref[1]) * dinv[:, None]
    z = jnp.dot(t, w1_ref[...], preferred_element_type=jnp.float32)
    z = jnp.maximum(z + b1_ref[...], 0.0)
    h = jnp.dot(z, w2_ref[...], preferred_element_type=jnp.float32) + b2_ref[...]
    o_ref[...] = jnp.where(_rowmask(i), h * dinv[:, None], 0.0)


def _tc_final_body(p_ref, degp_ref, w1_ref, b1_ref, w2_ref, b2_ref,
                   g_ref, bb_ref, wo_ref, bo_ref, o_ref):
    dinv = _dinv_of(degp_ref[...])
    t = (p_ref[0] + p_ref[1]) * dinv[:, None]
    z = jnp.dot(t, w1_ref[...], preferred_element_type=jnp.float32)
    z = jnp.maximum(z + b1_ref[...], 0.0)
    h = jnp.dot(z, w2_ref[...], preferred_element_type=jnp.float32) + b2_ref[...]
    mu = jnp.mean(h, axis=-1, keepdims=True)
    var = jnp.mean((h - mu) ** 2, axis=-1, keepdims=True)
    hn = (h - mu) * lax.rsqrt(var + EPS) * g_ref[...] + bb_ref[...]
    o_ref[...] = jnp.dot(hn, wo_ref[...], preferred_element_type=jnp.float32) + bo_ref[...]


def _vec_spec():
    return pl.BlockSpec((1, D), lambda i: (0, 0))


def _mat_spec():
    return pl.BlockSpec((D, D), lambda i: (0, 0))


def _row_spec():
    return pl.BlockSpec((BR, D), lambda i: (i, 0))


def _degp_spec():
    return pl.BlockSpec((NW, BR), lambda i: (0, i))


def _part_spec():
    return pl.BlockSpec((NC, BR, D), lambda i: (0, i, 0))


_GRID = NP // BR

_tc_in = pl.pallas_call(
    _tc_in_body,
    grid=(_GRID,),
    in_specs=[_row_spec(), _mat_spec(), _vec_spec(), _degp_spec()],
    out_specs=_row_spec(),
    out_shape=jax.ShapeDtypeStruct((NP, D), jnp.float32),
)

_tc_layer = pl.pallas_call(
    _tc_layer_body,
    grid=(_GRID,),
    in_specs=[_part_spec(), _degp_spec(), _mat_spec(), _vec_spec(),
              _mat_spec(), _vec_spec()],
    out_specs=_row_spec(),
    out_shape=jax.ShapeDtypeStruct((NP, D), jnp.float32),
)

_tc_final = pl.pallas_call(
    _tc_final_body,
    grid=(_GRID,),
    in_specs=[_part_spec(), _degp_spec(), _mat_spec(), _vec_spec(),
              _mat_spec(), _vec_spec(), _vec_spec(), _vec_spec(),
              _mat_spec(), _vec_spec()],
    out_specs=_row_spec(),
    out_shape=jax.ShapeDtypeStruct((NP, D), jnp.float32),
)


def kernel(x, edge_index, W_in, b_in, W1_0, b1_0, W2_0, b2_0,
           W1_1, b1_1, W2_1, b2_1, ln_g, ln_b, W_out, b_out):
    src = edge_index[0]
    dst = edge_index[1]
    pad = jnp.full((EPAD - E,), N, jnp.int32)
    src_p = jnp.concatenate([src, pad]).reshape(NW, NCHUNK, CH)
    dst_p = jnp.concatenate([dst, pad]).reshape(NW, NCHUNK, CH)
    x_p = jnp.pad(x, ((0, NP - N), (0, 0)))

    degp = _deg_call(dst_p)
    hs0 = _tc_in(x_p, W_in, b_in.reshape(1, D), degp)
    p0 = _agg_call(hs0, src_p, dst_p)
    hs1 = _tc_layer(p0, degp, W1_0, b1_0.reshape(1, D), W2_0, b2_0.reshape(1, D))
    p1 = _agg_call(hs1, src_p, dst_p)
    out = _tc_final(p1, degp, W1_1, b1_1.reshape(1, D), W2_1, b2_1.reshape(1, D),
                    ln_g.reshape(1, D), ln_b.reshape(1, D), W_out, b_out.reshape(1, D))
    return out[:N]


# speedup vs baseline: 18.6427x; 1.5712x over previous
import functools

import jax
import jax.numpy as jnp
from jax import lax
from jax.experimental import pallas as pl
from jax.experimental.pallas import tpu as pltpu
from jax.experimental.pallas import tpu_sc as plsc

N = 10000
D = 128
NP = 10240
E = 320000
NC = 2
NS = 16
NW = NC * NS
CH = 128
TOT_CHUNK = E // CH
N0 = 1600
N1 = TOT_CHUNK - N0
EPT_DEG = E // NW
RPT = NP // NS
BR = 256
EPS = 1e-5

_mesh = plsc.VectorSubcoreMesh(core_axis_name="c", subcore_axis_name="s")
_sc_params = pltpu.CompilerParams(needs_layout_passes=False)


def _deg_body(dst_hbm, out_hbm, dstv, degv):
    c = lax.axis_index("c")
    s = lax.axis_index("s")
    wid = s * NC + c
    pltpu.sync_copy(dst_hbm.at[pl.ds(wid * EPT_DEG, EPT_DEG)], dstv)
    zeros16 = jnp.zeros((16,), jnp.float32)

    def zbody(i, carry):
        degv[pl.ds(i * 16, 16)] = zeros16
        return carry

    lax.fori_loop(0, NP // 16, zbody, 0)
    ones16 = jnp.ones((16,), jnp.float32)

    def ebody(j, carry):
        for k in range(5):
            idx = dstv[pl.ds(j * 80 + k * 16, 16)]
            plsc.addupdate_scatter(degv, [idx], ones16)
        return carry

    lax.fori_loop(0, EPT_DEG // 80, ebody, 0)
    pltpu.sync_copy(degv, out_hbm.at[wid])


_deg_call = functools.partial(
    pl.kernel,
    out_type=jax.ShapeDtypeStruct((NW, NP), jnp.float32),
    mesh=_mesh,
    compiler_params=_sc_params,
    scratch_types=[
        pltpu.VMEM((EPT_DEG,), jnp.int32),
        pltpu.VMEM((NP,), jnp.float32),
    ],
)(_deg_body)


def _range_of(c, s):
    k0, r0 = N0 // NS, N0 % NS
    k1, r1 = N1 // NS, N1 % NS
    base0 = s * k0 + jnp.minimum(s, r0)
    cnt0 = k0 + (s < r0).astype(jnp.int32)
    base1 = N0 + s * k1 + jnp.minimum(s, r1)
    cnt1 = k1 + (s < r1).astype(jnp.int32)
    base = jnp.where(c == 0, base0, base1)
    cnt = jnp.where(c == 0, cnt0, cnt1)
    return base, cnt


def _agg_body(hs_hbm, src_hbm, dst_hbm, out_hbm, sidx, didx, rows, acc_sh,
              isem, gsem):
    c = lax.axis_index("c")
    s = lax.axis_index("s")
    base, cnt = _range_of(c, s)
    zeros16 = jnp.zeros((16,), jnp.float32)

    def zbody(i, carry):
        for k in range(D // 16):
            rows[0, i, pl.ds(k * 16, 16)] = zeros16
        return carry

    lax.fori_loop(0, CH, zbody, 0)
    for q in range(RPT // CH):
        pltpu.sync_copy(rows.at[0], acc_sh.at[pl.ds(s * RPT + q * CH, CH)])
    plsc.subcore_barrier()

    for p in range(3):
        pltpu.async_copy(src_hbm.at[base + p], sidx.at[p], isem)
        pltpu.async_copy(dst_hbm.at[base + p], didx.at[p], isem)
    pltpu.make_async_copy(src_hbm.at[base], sidx.at[0], isem).wait()
    pltpu.make_async_copy(dst_hbm.at[base], didx.at[0], isem).wait()
    pltpu.async_copy(hs_hbm.at[sidx.at[0]], rows.at[0], gsem)

    def cbody(j, carry):
        b = lax.rem(j, 2)
        pltpu.make_async_copy(hs_hbm.at[sidx.at[lax.rem(j, 4)]],
                              rows.at[b], gsem).wait()

        @pl.when(j + 1 < cnt)
        def _():
            nxt = lax.rem(j + 1, 4)
            pltpu.make_async_copy(src_hbm.at[base + j + 1], sidx.at[nxt],
                                  isem).wait()
            pltpu.make_async_copy(dst_hbm.at[base + j + 1], didx.at[nxt],
                                  isem).wait()
            pltpu.async_copy(hs_hbm.at[sidx.at[nxt]], rows.at[1 - b], gsem)

        @pl.when(j + 3 < cnt)
        def _():
            pltpu.async_copy(src_hbm.at[base + j + 3],
                             sidx.at[lax.rem(j + 3, 4)], isem)
            pltpu.async_copy(dst_hbm.at[base + j + 3],
                             didx.at[lax.rem(j + 3, 4)], isem)

        pltpu.sync_copy(rows.at[b], acc_sh.at[didx.at[lax.rem(j, 4)]], add=True)
        return carry

    lax.fori_loop(0, cnt, cbody, 0)
    plsc.subcore_barrier()
    pltpu.sync_copy(acc_sh.at[pl.ds(s * RPT, RPT)],
                    out_hbm.at[c, pl.ds(s * RPT, RPT)])


_agg_call = functools.partial(
    pl.kernel,
    out_type=jax.ShapeDtypeStruct((NC, NP, D), jnp.float32),
    mesh=_mesh,
    compiler_params=_sc_params,
    scratch_types=[
        pltpu.VMEM((4, CH), jnp.int32),
        pltpu.VMEM((4, CH), jnp.int32),
        pltpu.VMEM((2, CH, D), jnp.float32),
        pltpu.VMEM_SHARED((NP, D), jnp.float32),
        pltpu.SemaphoreType.DMA,
        pltpu.SemaphoreType.DMA,
    ],
)(_agg_body)


def _rowmask(i):
    rid = lax.broadcasted_iota(jnp.int32, (BR, 1), 0) + i * BR
    return rid < N


def _tc_in_body(x_ref, w_ref, b_ref, degp_ref, o_ref, dinv_ref):
    i = pl.program_id(0)
    dinv = lax.rsqrt(jnp.maximum(jnp.sum(degp_ref[...], axis=0), 1.0))
    dinv_ref[...] = dinv[None, :]
    h = jnp.dot(x_ref[...], w_ref[...], preferred_element_type=jnp.float32)
    h = jnp.maximum(h + b_ref[...], 0.0)
    o_ref[...] = jnp.where(_rowmask(i), h * dinv[:, None], 0.0)


def _tc_layer_body(p_ref, dinv_ref, w1_ref, b1_ref, w2_ref, b2_ref, o_ref):
    i = pl.program_id(0)
    dinv = dinv_ref[0]
    t = (p_ref[0] + p_ref[1]) * dinv[:, None]
    z = jnp.dot(t, w1_ref[...], preferred_element_type=jnp.float32)
    z = jnp.maximum(z + b1_ref[...], 0.0)
    h = jnp.dot(z, w2_ref[...], preferred_element_type=jnp.float32) + b2_ref[...]
    o_ref[...] = jnp.where(_rowmask(i), h * dinv[:, None], 0.0)


def _tc_final_body(p_ref, dinv_ref, w1_ref, b1_ref, w2_ref, b2_ref,
                   g_ref, bb_ref, wo_ref, bo_ref, o_ref):
    dinv = dinv_ref[0]
    t = (p_ref[0] + p_ref[1]) * dinv[:, None]
    z = jnp.dot(t, w1_ref[...], preferred_element_type=jnp.float32)
    z = jnp.maximum(z + b1_ref[...], 0.0)
    h = jnp.dot(z, w2_ref[...], preferred_element_type=jnp.float32) + b2_ref[...]
    mu = jnp.mean(h, axis=-1, keepdims=True)
    var = jnp.mean((h - mu) ** 2, axis=-1, keepdims=True)
    hn = (h - mu) * lax.rsqrt(var + EPS) * g_ref[...] + bb_ref[...]
    o_ref[...] = jnp.dot(hn, wo_ref[...], preferred_element_type=jnp.float32) + bo_ref[...]


def _vec_spec():
    return pl.BlockSpec((1, D), lambda i: (0, 0))


def _mat_spec():
    return pl.BlockSpec((D, D), lambda i: (0, 0))


def _row_spec():
    return pl.BlockSpec((BR, D), lambda i: (i, 0))


def _dinv_spec():
    return pl.BlockSpec((1, BR), lambda i: (0, i))


def _part_spec():
    return pl.BlockSpec((NC, BR, D), lambda i: (0, i, 0))


_GRID = NP // BR

_tc_in = pl.pallas_call(
    _tc_in_body,
    grid=(_GRID,),
    in_specs=[_row_spec(), _mat_spec(), _vec_spec(),
              pl.BlockSpec((NW, BR), lambda i: (0, i))],
    out_specs=[_row_spec(), _dinv_spec()],
    out_shape=[jax.ShapeDtypeStruct((NP, D), jnp.float32),
               jax.ShapeDtypeStruct((1, NP), jnp.float32)],
)

_tc_layer = pl.pallas_call(
    _tc_layer_body,
    grid=(_GRID,),
    in_specs=[_part_spec(), _dinv_spec(), _mat_spec(), _vec_spec(),
              _mat_spec(), _vec_spec()],
    out_specs=_row_spec(),
    out_shape=jax.ShapeDtypeStruct((NP, D), jnp.float32),
)

_tc_final = pl.pallas_call(
    _tc_final_body,
    grid=(_GRID,),
    in_specs=[_part_spec(), _dinv_spec(), _mat_spec(), _vec_spec(),
              _mat_spec(), _vec_spec(), _vec_spec(), _vec_spec(),
              _mat_spec(), _vec_spec()],
    out_specs=_row_spec(),
    out_shape=jax.ShapeDtypeStruct((NP, D), jnp.float32),
)


def kernel(x, edge_index, W_in, b_in, W1_0, b1_0, W2_0, b2_0,
           W1_1, b1_1, W2_1, b2_1, ln_g, ln_b, W_out, b_out):
    src = edge_index[0]
    dst = edge_index[1]
    src_p = src.reshape(TOT_CHUNK, CH)
    dst_p = dst.reshape(TOT_CHUNK, CH)
    x_p = jnp.pad(x, ((0, NP - N), (0, 0)))

    degp = _deg_call(dst)
    hs0, dinv = _tc_in(x_p, W_in, b_in.reshape(1, D), degp)
    p0 = _agg_call(hs0, src_p, dst_p)
    hs1 = _tc_layer(p0, dinv, W1_0, b1_0.reshape(1, D), W2_0, b2_0.reshape(1, D))
    p1 = _agg_call(hs1, src_p, dst_p)
    out = _tc_final(p1, dinv, W1_1, b1_1.reshape(1, D), W2_1, b2_1.reshape(1, D),
                    ln_g.reshape(1, D), ln_b.reshape(1, D), W_out, b_out.reshape(1, D))
    return out[:N]


# speedup vs baseline: 21.4107x; 1.1485x over previous
import functools

import jax
import jax.numpy as jnp
from jax import lax
from jax.experimental import pallas as pl
from jax.experimental.pallas import tpu as pltpu
from jax.experimental.pallas import tpu_sc as plsc

N = 10000
D = 128
NP = 10240
E = 320000
NC = 2
NS = 16
NW = NC * NS
CH = 128
TOT_CHUNK = E // CH
N0 = 1250
N1 = TOT_CHUNK - N0
EPT_DEG = E // NW
RPT = NP // NS
BR = 256
EPS = 1e-5

_mesh = plsc.VectorSubcoreMesh(core_axis_name="c", subcore_axis_name="s")
_sc_params = pltpu.CompilerParams(needs_layout_passes=False)


def _deg_body(dst_hbm, out_hbm, dstv, degv):
    c = lax.axis_index("c")
    s = lax.axis_index("s")
    wid = s * NC + c
    pltpu.sync_copy(dst_hbm.at[pl.ds(wid * EPT_DEG, EPT_DEG)], dstv)
    zeros16 = jnp.zeros((16,), jnp.float32)

    def zbody(i, carry):
        degv[pl.ds(i * 16, 16)] = zeros16
        return carry

    lax.fori_loop(0, NP // 16, zbody, 0)
    ones16 = jnp.ones((16,), jnp.float32)

    def ebody(j, carry):
        for k in range(5):
            idx = dstv[pl.ds(j * 80 + k * 16, 16)]
            plsc.addupdate_scatter(degv, [idx], ones16)
        return carry

    lax.fori_loop(0, EPT_DEG // 80, ebody, 0)
    pltpu.sync_copy(degv, out_hbm.at[wid])


_deg_call = functools.partial(
    pl.kernel,
    out_type=jax.ShapeDtypeStruct((NW, NP), jnp.float32),
    mesh=_mesh,
    compiler_params=_sc_params,
    scratch_types=[
        pltpu.VMEM((EPT_DEG,), jnp.int32),
        pltpu.VMEM((NP,), jnp.float32),
    ],
)(_deg_body)


def _range_of(c, s):
    k0, r0 = N0 // NS, N0 % NS
    k1, r1 = N1 // NS, N1 % NS
    base0 = s * k0 + jnp.minimum(s, r0)
    cnt0 = k0 + (s < r0).astype(jnp.int32)
    base1 = N0 + s * k1 + jnp.minimum(s, r1)
    cnt1 = k1 + (s < r1).astype(jnp.int32)
    base = jnp.where(c == 0, base0, base1)
    cnt = jnp.where(c == 0, cnt0, cnt1)
    return base, cnt


def _agg_body(hs_hbm, src_hbm, dst_hbm, out_hbm, sidx, didx, rows, acc_sh,
              isem, gsem):
    c = lax.axis_index("c")
    s = lax.axis_index("s")
    base, cnt = _range_of(c, s)
    zeros16 = jnp.zeros((16,), jnp.float32)

    def zbody(i, carry):
        for k in range(D // 16):
            rows[0, i, pl.ds(k * 16, 16)] = zeros16
        return carry

    lax.fori_loop(0, CH, zbody, 0)
    for q in range(RPT // CH):
        pltpu.sync_copy(rows.at[0], acc_sh.at[pl.ds(s * RPT + q * CH, CH)])
    plsc.subcore_barrier()

    for p in range(3):
        pltpu.async_copy(src_hbm.at[base + p], sidx.at[p], isem)
        pltpu.async_copy(dst_hbm.at[base + p], didx.at[p], isem)
    pltpu.make_async_copy(src_hbm.at[base], sidx.at[0], isem).wait()
    pltpu.make_async_copy(dst_hbm.at[base], didx.at[0], isem).wait()
    pltpu.async_copy(hs_hbm.at[sidx.at[0]], rows.at[0], gsem)

    def cbody(j, carry):
        b = lax.rem(j, 2)
        pltpu.make_async_copy(hs_hbm.at[sidx.at[lax.rem(j, 4)]],
                              rows.at[b], gsem).wait()

        @pl.when(j + 1 < cnt)
        def _():
            nxt = lax.rem(j + 1, 4)
            pltpu.make_async_copy(src_hbm.at[base + j + 1], sidx.at[nxt],
                                  isem).wait()
            pltpu.make_async_copy(dst_hbm.at[base + j + 1], didx.at[nxt],
                                  isem).wait()
            pltpu.async_copy(hs_hbm.at[sidx.at[nxt]], rows.at[1 - b], gsem)

        @pl.when(j + 3 < cnt)
        def _():
            pltpu.async_copy(src_hbm.at[base + j + 3],
                             sidx.at[lax.rem(j + 3, 4)], isem)
            pltpu.async_copy(dst_hbm.at[base + j + 3],
                             didx.at[lax.rem(j + 3, 4)], isem)

        pltpu.sync_copy(rows.at[b], acc_sh.at[didx.at[lax.rem(j, 4)]], add=True)
        return carry

    lax.fori_loop(0, cnt, cbody, 0)
    plsc.subcore_barrier()
    pltpu.sync_copy(acc_sh.at[pl.ds(s * RPT, RPT)],
                    out_hbm.at[c, pl.ds(s * RPT, RPT)])


_agg_call = functools.partial(
    pl.kernel,
    out_type=jax.ShapeDtypeStruct((NC, NP, D), jnp.float32),
    mesh=_mesh,
    compiler_params=_sc_params,
    scratch_types=[
        pltpu.VMEM((4, CH), jnp.int32),
        pltpu.VMEM((4, CH), jnp.int32),
        pltpu.VMEM((2, CH, D), jnp.float32),
        pltpu.VMEM_SHARED((NP, D), jnp.float32),
        pltpu.SemaphoreType.DMA,
        pltpu.SemaphoreType.DMA,
    ],
)(_agg_body)


def _rowmask(i):
    rid = lax.broadcasted_iota(jnp.int32, (BR, 1), 0) + i * BR
    return rid < N


def _tc_in_body(x_ref, w_ref, b_ref, degp_ref, o_ref, dinv_ref):
    i = pl.program_id(0)
    dinv = lax.rsqrt(jnp.maximum(jnp.sum(degp_ref[...], axis=0), 1.0))
    dinv_ref[...] = dinv[None, :]
    h = jnp.dot(x_ref[...], w_ref[...], preferred_element_type=jnp.float32)
    h = jnp.maximum(h + b_ref[...], 0.0)
    o_ref[...] = jnp.where(_rowmask(i), h * dinv[:, None], 0.0)


def _tc_layer_body(p_ref, dinv_ref, w1_ref, b1_ref, w2_ref, b2_ref, o_ref):
    i = pl.program_id(0)
    dinv = dinv_ref[0]
    t = (p_ref[0] + p_ref[1]) * dinv[:, None]
    z = jnp.dot(t, w1_ref[...], preferred_element_type=jnp.float32)
    z = jnp.maximum(z + b1_ref[...], 0.0)
    h = jnp.dot(z, w2_ref[...], preferred_element_type=jnp.float32) + b2_ref[...]
    o_ref[...] = jnp.where(_rowmask(i), h * dinv[:, None], 0.0)


def _tc_final_body(p_ref, dinv_ref, w1_ref, b1_ref, w2_ref, b2_ref,
                   g_ref, bb_ref, wo_ref, bo_ref, o_ref):
    dinv = dinv_ref[0]
    t = (p_ref[0] + p_ref[1]) * dinv[:, None]
    z = jnp.dot(t, w1_ref[...], preferred_element_type=jnp.float32)
    z = jnp.maximum(z + b1_ref[...], 0.0)
    h = jnp.dot(z, w2_ref[...], preferred_element_type=jnp.float32) + b2_ref[...]
    mu = jnp.mean(h, axis=-1, keepdims=True)
    var = jnp.mean((h - mu) ** 2, axis=-1, keepdims=True)
    hn = (h - mu) * lax.rsqrt(var + EPS) * g_ref[...] + bb_ref[...]
    o_ref[...] = jnp.dot(hn, wo_ref[...], preferred_element_type=jnp.float32) + bo_ref[...]


def _vec_spec():
    return pl.BlockSpec((1, D), lambda i: (0, 0))


def _mat_spec():
    return pl.BlockSpec((D, D), lambda i: (0, 0))


def _row_spec():
    return pl.BlockSpec((BR, D), lambda i: (i, 0))


def _dinv_spec():
    return pl.BlockSpec((1, BR), lambda i: (0, i))


def _part_spec():
    return pl.BlockSpec((NC, BR, D), lambda i: (0, i, 0))


_GRID = NP // BR

_tc_in = pl.pallas_call(
    _tc_in_body,
    grid=(_GRID,),
    in_specs=[_row_spec(), _mat_spec(), _vec_spec(),
              pl.BlockSpec((NW, BR), lambda i: (0, i))],
    out_specs=[_row_spec(), _dinv_spec()],
    out_shape=[jax.ShapeDtypeStruct((NP, D), jnp.float32),
               jax.ShapeDtypeStruct((1, NP), jnp.float32)],
)

_tc_layer = pl.pallas_call(
    _tc_layer_body,
    grid=(_GRID,),
    in_specs=[_part_spec(), _dinv_spec(), _mat_spec(), _vec_spec(),
              _mat_spec(), _vec_spec()],
    out_specs=_row_spec(),
    out_shape=jax.ShapeDtypeStruct((NP, D), jnp.float32),
)

_tc_final = pl.pallas_call(
    _tc_final_body,
    grid=(_GRID,),
    in_specs=[_part_spec(), _dinv_spec(), _mat_spec(), _vec_spec(),
              _mat_spec(), _vec_spec(), _vec_spec(), _vec_spec(),
              _mat_spec(), _vec_spec()],
    out_specs=_row_spec(),
    out_shape=jax.ShapeDtypeStruct((NP, D), jnp.float32),
)


def kernel(x, edge_index, W_in, b_in, W1_0, b1_0, W2_0, b2_0,
           W1_1, b1_1, W2_1, b2_1, ln_g, ln_b, W_out, b_out):
    src = edge_index[0]
    dst = edge_index[1]
    src_p = src.reshape(TOT_CHUNK, CH)
    dst_p = dst.reshape(TOT_CHUNK, CH)
    x_p = jnp.pad(x, ((0, NP - N), (0, 0)))

    degp = _deg_call(dst)
    hs0, dinv = _tc_in(x_p, W_in, b_in.reshape(1, D), degp)
    p0 = _agg_call(hs0, src_p, dst_p)
    hs1 = _tc_layer(p0, dinv, W1_0, b1_0.reshape(1, D), W2_0, b2_0.reshape(1, D))
    p1 = _agg_call(hs1, src_p, dst_p)
    out = _tc_final(p1, dinv, W1_1, b1_1.reshape(1, D), W2_1, b2_1.reshape(1, D),
                    ln_g.reshape(1, D), ln_b.reshape(1, D), W_out, b_out.reshape(1, D))
    return out[:N]


# speedup vs baseline: 29.7282x; 1.3885x over previous
import functools

import jax
import jax.numpy as jnp
from jax import lax
from jax.experimental import pallas as pl
from jax.experimental.pallas import tpu as pltpu
from jax.experimental.pallas import tpu_sc as plsc

N = 10000
D = 128
NP = 10240
E = 320000
NC = 2
NS = 16
NW = NC * NS
CH = 128
TOT_CHUNK = E // CH
N0 = 1250
N1 = TOT_CHUNK - N0
EPT_DEG = E // NW
ACCR = 10048
FRT = 632
LRT = ACCR - 15 * FRT
BR = 512
EPS = 1e-5

_mesh = plsc.VectorSubcoreMesh(core_axis_name="c", subcore_axis_name="s")
_sc_params = pltpu.CompilerParams(needs_layout_passes=False)


def _deg_body(dst_hbm, out_hbm, dstv, degv):
    c = lax.axis_index("c")
    s = lax.axis_index("s")
    wid = s * NC + c
    pltpu.sync_copy(dst_hbm.at[pl.ds(wid * EPT_DEG, EPT_DEG)], dstv)
    zeros16 = jnp.zeros((16,), jnp.float32)

    def zbody(i, carry):
        degv[pl.ds(i * 16, 16)] = zeros16
        return carry

    lax.fori_loop(0, NP // 16, zbody, 0)
    ones16 = jnp.ones((16,), jnp.float32)

    def ebody(j, carry):
        for k in range(5):
            idx = dstv[pl.ds(j * 80 + k * 16, 16)]
            plsc.addupdate_scatter(degv, [idx], ones16)
        return carry

    lax.fori_loop(0, EPT_DEG // 80, ebody, 0)
    pltpu.sync_copy(degv, out_hbm.at[wid])


_deg_call = functools.partial(
    pl.kernel,
    out_type=jax.ShapeDtypeStruct((NW, NP), jnp.float32),
    mesh=_mesh,
    compiler_params=_sc_params,
    scratch_types=[
        pltpu.VMEM((EPT_DEG,), jnp.int32),
        pltpu.VMEM((NP,), jnp.float32),
    ],
)(_deg_body)


def _range_of(c, s):
    k0, r0 = N0 // NS, N0 % NS
    k1, r1 = N1 // NS, N1 % NS
    base0 = s * k0 + jnp.minimum(s, r0)
    cnt0 = k0 + (s < r0).astype(jnp.int32)
    base1 = N0 + s * k1 + jnp.minimum(s, r1)
    cnt1 = k1 + (s < r1).astype(jnp.int32)
    base = jnp.where(c == 0, base0, base1)
    cnt = jnp.where(c == 0, cnt0, cnt1)
    return base, cnt


def _agg_body(hs_hbm, src_hbm, dst_hbm, out_hbm, sidx, didx, rows, acc_sh,
              isem, gsem, ssem):
    c = lax.axis_index("c")
    s = lax.axis_index("s")
    base, cnt = _range_of(c, s)
    zeros16 = jnp.zeros((16,), jnp.float32)

    def zbody(i, carry):
        for k in range(D // 16):
            rows[0, i, pl.ds(k * 16, 16)] = zeros16
        return carry

    lax.fori_loop(0, CH, zbody, 0)
    for q in range(4):
        pltpu.sync_copy(rows.at[0], acc_sh.at[pl.ds(s * FRT + q * CH, CH)])

    @pl.when(s < NS - 1)
    def _():
        pltpu.sync_copy(rows.at[0, pl.ds(0, FRT - 4 * CH)],
                        acc_sh.at[pl.ds(s * FRT + 4 * CH, FRT - 4 * CH)])

    @pl.when(s == NS - 1)
    def _():
        pltpu.sync_copy(rows.at[0, pl.ds(0, LRT - 4 * CH)],
                        acc_sh.at[pl.ds(s * FRT + 4 * CH, LRT - 4 * CH)])

    plsc.subcore_barrier()

    for p in range(3):
        pltpu.async_copy(src_hbm.at[base + p], sidx.at[p], isem.at[p])
        pltpu.async_copy(dst_hbm.at[base + p], didx.at[p], isem.at[p])
    pltpu.make_async_copy(src_hbm.at[base], sidx.at[0], isem.at[0]).wait()
    pltpu.make_async_copy(dst_hbm.at[base], didx.at[0], isem.at[0]).wait()
    pltpu.async_copy(hs_hbm.at[sidx.at[0]], rows.at[0], gsem.at[0])
    pltpu.make_async_copy(src_hbm.at[base + 1], sidx.at[1], isem.at[1]).wait()
    pltpu.make_async_copy(dst_hbm.at[base + 1], didx.at[1], isem.at[1]).wait()
    pltpu.async_copy(hs_hbm.at[sidx.at[1]], rows.at[1], gsem.at[1])

    def cbody(j, carry):
        b = lax.rem(j, 3)
        slot = lax.rem(j, 4)
        par = lax.rem(j, 2)
        pltpu.make_async_copy(hs_hbm.at[sidx.at[slot]], rows.at[b],
                              gsem.at[par]).wait()

        @pl.when(j >= 1)
        def _():
            pltpu.make_async_copy(rows.at[lax.rem(j + 2, 3)],
                                  acc_sh.at[didx.at[lax.rem(j + 3, 4)]],
                                  ssem).wait()

        @pl.when(j + 2 < cnt)
        def _():
            n2 = lax.rem(j + 2, 4)
            pltpu.make_async_copy(src_hbm.at[base + j + 2], sidx.at[n2],
                                  isem.at[n2]).wait()
            pltpu.make_async_copy(dst_hbm.at[base + j + 2], didx.at[n2],
                                  isem.at[n2]).wait()
            pltpu.async_copy(hs_hbm.at[sidx.at[n2]], rows.at[lax.rem(j + 2, 3)],
                             gsem.at[par])

        @pl.when(j + 3 < cnt)
        def _():
            n3 = lax.rem(j + 3, 4)
            pltpu.async_copy(src_hbm.at[base + j + 3], sidx.at[n3], isem.at[n3])
            pltpu.async_copy(dst_hbm.at[base + j + 3], didx.at[n3], isem.at[n3])

        pltpu.async_copy(rows.at[b], acc_sh.at[didx.at[slot]], ssem, add=True)
        return carry

    lax.fori_loop(0, cnt, cbody, 0)
    pltpu.make_async_copy(rows.at[lax.rem(cnt - 1, 3)],
                          acc_sh.at[didx.at[lax.rem(cnt - 1, 4)]], ssem).wait()
    plsc.subcore_barrier()

    @pl.when(s < NS - 1)
    def _():
        pltpu.sync_copy(acc_sh.at[pl.ds(s * FRT, FRT)],
                        out_hbm.at[c, pl.ds(s * FRT, FRT)])

    @pl.when(s == NS - 1)
    def _():
        pltpu.sync_copy(acc_sh.at[pl.ds(s * FRT, LRT)],
                        out_hbm.at[c, pl.ds(s * FRT, LRT)])


_agg_call = functools.partial(
    pl.kernel,
    out_type=jax.ShapeDtypeStruct((NC, ACCR, D), jnp.float32),
    mesh=_mesh,
    compiler_params=_sc_params,
    scratch_types=[
        pltpu.VMEM((4, CH), jnp.int32),
        pltpu.VMEM((4, CH), jnp.int32),
        pltpu.VMEM((3, CH, D), jnp.float32),
        pltpu.VMEM_SHARED((ACCR, D), jnp.float32),
        pltpu.SemaphoreType.DMA((4,)),
        pltpu.SemaphoreType.DMA((2,)),
        pltpu.SemaphoreType.DMA,
    ],
)(_agg_body)


def _rowmask(i):
    rid = lax.broadcasted_iota(jnp.int32, (BR, 1), 0) + i * BR
    return rid < N


def _tc_in_body(x_ref, w_ref, b_ref, degp_ref, o_ref, dinv_ref):
    i = pl.program_id(0)
    dinv = lax.rsqrt(jnp.maximum(jnp.sum(degp_ref[...], axis=0), 1.0))
    dinv_ref[...] = dinv[None, :]
    h = jnp.dot(x_ref[...], w_ref[...], preferred_element_type=jnp.float32)
    h = jnp.maximum(h + b_ref[...], 0.0)
    o_ref[...] = jnp.where(_rowmask(i), h * dinv[:, None], 0.0)


def _tc_layer_body(p_ref, dinv_ref, w1_ref, b1_ref, w2_ref, b2_ref, o_ref):
    i = pl.program_id(0)
    dinv = dinv_ref[0]
    t = (p_ref[0] + p_ref[1]) * dinv[:, None]
    z = jnp.dot(t, w1_ref[...], preferred_element_type=jnp.float32)
    z = jnp.maximum(z + b1_ref[...], 0.0)
    h = jnp.dot(z, w2_ref[...], preferred_element_type=jnp.float32) + b2_ref[...]
    o_ref[...] = jnp.where(_rowmask(i), h * dinv[:, None], 0.0)


def _tc_final_body(p_ref, dinv_ref, w1_ref, b1_ref, w2_ref, b2_ref,
                   g_ref, bb_ref, wo_ref, bo_ref, o_ref):
    dinv = dinv_ref[0]
    t = (p_ref[0] + p_ref[1]) * dinv[:, None]
    z = jnp.dot(t, w1_ref[...], preferred_element_type=jnp.float32)
    z = jnp.maximum(z + b1_ref[...], 0.0)
    h = jnp.dot(z, w2_ref[...], preferred_element_type=jnp.float32) + b2_ref[...]
    mu = jnp.mean(h, axis=-1, keepdims=True)
    var = jnp.mean((h - mu) ** 2, axis=-1, keepdims=True)
    hn = (h - mu) * lax.rsqrt(var + EPS) * g_ref[...] + bb_ref[...]
    o_ref[...] = jnp.dot(hn, wo_ref[...], preferred_element_type=jnp.float32) + bo_ref[...]


def _vec_spec():
    return pl.BlockSpec((1, D), lambda i: (0, 0))


def _mat_spec():
    return pl.BlockSpec((D, D), lambda i: (0, 0))


def _row_spec():
    return pl.BlockSpec((BR, D), lambda i: (i, 0))


def _dinv_spec():
    return pl.BlockSpec((1, BR), lambda i: (0, i))


def _part_spec():
    return pl.BlockSpec((NC, BR, D), lambda i: (0, i, 0))


_GRID = NP // BR

_tc_in = pl.pallas_call(
    _tc_in_body,
    grid=(_GRID,),
    in_specs=[_row_spec(), _mat_spec(), _vec_spec(),
              pl.BlockSpec((NW, BR), lambda i: (0, i))],
    out_specs=[_row_spec(), _dinv_spec()],
    out_shape=[jax.ShapeDtypeStruct((NP, D), jnp.float32),
               jax.ShapeDtypeStruct((1, NP), jnp.float32)],
)

_tc_layer = pl.pallas_call(
    _tc_layer_body,
    grid=(_GRID,),
    in_specs=[_part_spec(), _dinv_spec(), _mat_spec(), _vec_spec(),
              _mat_spec(), _vec_spec()],
    out_specs=_row_spec(),
    out_shape=jax.ShapeDtypeStruct((NP, D), jnp.float32),
)

_tc_final = pl.pallas_call(
    _tc_final_body,
    grid=(_GRID,),
    in_specs=[_part_spec(), _dinv_spec(), _mat_spec(), _vec_spec(),
              _mat_spec(), _vec_spec(), _vec_spec(), _vec_spec(),
              _mat_spec(), _vec_spec()],
    out_specs=_row_spec(),
    out_shape=jax.ShapeDtypeStruct((NP, D), jnp.float32),
)


def kernel(x, edge_index, W_in, b_in, W1_0, b1_0, W2_0, b2_0,
           W1_1, b1_1, W2_1, b2_1, ln_g, ln_b, W_out, b_out):
    src = edge_index[0]
    dst = edge_index[1]
    src_p = src.reshape(TOT_CHUNK, CH)
    dst_p = dst.reshape(TOT_CHUNK, CH)

    degp = _deg_call(dst)
    hs0, dinv = _tc_in(x, W_in, b_in.reshape(1, D), degp)
    p0 = _agg_call(hs0, src_p, dst_p)
    hs1 = _tc_layer(p0, dinv, W1_0, b1_0.reshape(1, D), W2_0, b2_0.reshape(1, D))
    p1 = _agg_call(hs1, src_p, dst_p)
    out = _tc_final(p1, dinv, W1_1, b1_1.reshape(1, D), W2_1, b2_1.reshape(1, D),
                    ln_g.reshape(1, D), ln_b.reshape(1, D), W_out, b_out.reshape(1, D))
    return out[:N]


# speedup vs baseline: 31.6906x; 1.0660x over previous
import functools

import jax
import jax.numpy as jnp
from jax import lax
from jax.experimental import pallas as pl
from jax.experimental.pallas import tpu as pltpu
from jax.experimental.pallas import tpu_sc as plsc

N = 10000
D = 128
NP = 10240
E = 320000
NC = 2
NS = 16
NW = NC * NS
CH = 128
TOT_CHUNK = E // CH
N0 = 1250
N1 = TOT_CHUNK - N0
EPT_DEG = E // NW
ACCR = 10048
FRT = 632
LRT = ACCR - 15 * FRT
BR = 1024
EPS = 1e-5

_mesh = plsc.VectorSubcoreMesh(core_axis_name="c", subcore_axis_name="s")
_sc_params = pltpu.CompilerParams(needs_layout_passes=False)


def _deg_body(dst_hbm, out_hbm, dstv, degv):
    c = lax.axis_index("c")
    s = lax.axis_index("s")
    wid = s * NC + c
    pltpu.sync_copy(dst_hbm.at[pl.ds(wid * EPT_DEG, EPT_DEG)], dstv)
    zeros16 = jnp.zeros((16,), jnp.float32)

    def zbody(i, carry):
        degv[pl.ds(i * 16, 16)] = zeros16
        return carry

    lax.fori_loop(0, NP // 16, zbody, 0)
    ones16 = jnp.ones((16,), jnp.float32)

    def ebody(j, carry):
        for k in range(5):
            idx = dstv[pl.ds(j * 80 + k * 16, 16)]
            plsc.addupdate_scatter(degv, [idx], ones16)
        return carry

    lax.fori_loop(0, EPT_DEG // 80, ebody, 0)
    pltpu.sync_copy(degv, out_hbm.at[wid])


_deg_call = functools.partial(
    pl.kernel,
    out_type=jax.ShapeDtypeStruct((NW, NP), jnp.float32),
    mesh=_mesh,
    compiler_params=_sc_params,
    scratch_types=[
        pltpu.VMEM((EPT_DEG,), jnp.int32),
        pltpu.VMEM((NP,), jnp.float32),
    ],
)(_deg_body)


def _range_of(c, s):
    k0, r0 = N0 // NS, N0 % NS
    k1, r1 = N1 // NS, N1 % NS
    base0 = s * k0 + jnp.minimum(s, r0)
    cnt0 = k0 + (s < r0).astype(jnp.int32)
    base1 = N0 + s * k1 + jnp.minimum(s, r1)
    cnt1 = k1 + (s < r1).astype(jnp.int32)
    base = jnp.where(c == 0, base0, base1)
    cnt = jnp.where(c == 0, cnt0, cnt1)
    return base, cnt


def _agg_body(hs_hbm, src_hbm, dst_hbm, out_hbm, sidx, didx, rows, acc_sh,
              isem, gsem, ssem):
    c = lax.axis_index("c")
    s = lax.axis_index("s")
    base, cnt = _range_of(c, s)
    zeros16 = jnp.zeros((16,), jnp.float32)

    def zbody(i, carry):
        for k in range(D // 16):
            rows[0, i, pl.ds(k * 16, 16)] = zeros16
        return carry

    lax.fori_loop(0, CH, zbody, 0)
    for q in range(4):
        pltpu.sync_copy(rows.at[0], acc_sh.at[pl.ds(s * FRT + q * CH, CH)])

    @pl.when(s < NS - 1)
    def _():
        pltpu.sync_copy(rows.at[0, pl.ds(0, FRT - 4 * CH)],
                        acc_sh.at[pl.ds(s * FRT + 4 * CH, FRT - 4 * CH)])

    @pl.when(s == NS - 1)
    def _():
        pltpu.sync_copy(rows.at[0, pl.ds(0, LRT - 4 * CH)],
                        acc_sh.at[pl.ds(s * FRT + 4 * CH, LRT - 4 * CH)])

    plsc.subcore_barrier()

    for p in range(3):
        pltpu.async_copy(src_hbm.at[base + p], sidx.at[p], isem.at[p])
        pltpu.async_copy(dst_hbm.at[base + p], didx.at[p], isem.at[p])
    pltpu.make_async_copy(src_hbm.at[base], sidx.at[0], isem.at[0]).wait()
    pltpu.make_async_copy(dst_hbm.at[base], didx.at[0], isem.at[0]).wait()
    pltpu.async_copy(hs_hbm.at[sidx.at[0]], rows.at[0], gsem.at[0])
    pltpu.make_async_copy(src_hbm.at[base + 1], sidx.at[1], isem.at[1]).wait()
    pltpu.make_async_copy(dst_hbm.at[base + 1], didx.at[1], isem.at[1]).wait()
    pltpu.async_copy(hs_hbm.at[sidx.at[1]], rows.at[1], gsem.at[1])

    def cbody(j, carry):
        b = lax.rem(j, 3)
        slot = lax.rem(j, 4)
        par = lax.rem(j, 2)
        pltpu.make_async_copy(hs_hbm.at[sidx.at[slot]], rows.at[b],
                              gsem.at[par]).wait()

        @pl.when(j >= 1)
        def _():
            pltpu.make_async_copy(rows.at[lax.rem(j + 2, 3)],
                                  acc_sh.at[didx.at[lax.rem(j + 3, 4)]],
                                  ssem).wait()

        @pl.when(j + 2 < cnt)
        def _():
            n2 = lax.rem(j + 2, 4)
            pltpu.make_async_copy(src_hbm.at[base + j + 2], sidx.at[n2],
                                  isem.at[n2]).wait()
            pltpu.make_async_copy(dst_hbm.at[base + j + 2], didx.at[n2],
                                  isem.at[n2]).wait()
            pltpu.async_copy(hs_hbm.at[sidx.at[n2]], rows.at[lax.rem(j + 2, 3)],
                             gsem.at[par])

        @pl.when(j + 3 < cnt)
        def _():
            n3 = lax.rem(j + 3, 4)
            pltpu.async_copy(src_hbm.at[base + j + 3], sidx.at[n3], isem.at[n3])
            pltpu.async_copy(dst_hbm.at[base + j + 3], didx.at[n3], isem.at[n3])

        pltpu.async_copy(rows.at[b], acc_sh.at[didx.at[slot]], ssem, add=True)
        return carry

    lax.fori_loop(0, cnt, cbody, 0)
    pltpu.make_async_copy(rows.at[lax.rem(cnt - 1, 3)],
                          acc_sh.at[didx.at[lax.rem(cnt - 1, 4)]], ssem).wait()
    plsc.subcore_barrier()

    @pl.when(s < NS - 1)
    def _():
        pltpu.sync_copy(acc_sh.at[pl.ds(s * FRT, FRT)],
                        out_hbm.at[c, pl.ds(s * FRT, FRT)])

    @pl.when(s == NS - 1)
    def _():
        pltpu.sync_copy(acc_sh.at[pl.ds(s * FRT, LRT)],
                        out_hbm.at[c, pl.ds(s * FRT, LRT)])


_agg_call = functools.partial(
    pl.kernel,
    out_type=jax.ShapeDtypeStruct((NC, ACCR, D), jnp.float32),
    mesh=_mesh,
    compiler_params=_sc_params,
    scratch_types=[
        pltpu.VMEM((4, CH), jnp.int32),
        pltpu.VMEM((4, CH), jnp.int32),
        pltpu.VMEM((3, CH, D), jnp.float32),
        pltpu.VMEM_SHARED((ACCR, D), jnp.float32),
        pltpu.SemaphoreType.DMA((4,)),
        pltpu.SemaphoreType.DMA((2,)),
        pltpu.SemaphoreType.DMA,
    ],
)(_agg_body)


def _rowmask(i):
    rid = lax.broadcasted_iota(jnp.int32, (BR, 1), 0) + i * BR
    return rid < N


def _tc_in_body(x_ref, w_ref, b_ref, degp_ref, o_ref, dinv_ref):
    i = pl.program_id(0)
    dinv = lax.rsqrt(jnp.maximum(jnp.sum(degp_ref[...], axis=0), 1.0))
    dinv_ref[...] = dinv[None, :]
    h = jnp.dot(x_ref[...], w_ref[...], preferred_element_type=jnp.float32)
    h = jnp.maximum(h + b_ref[...], 0.0)
    o_ref[...] = jnp.where(_rowmask(i), h * dinv[:, None], 0.0)


def _tc_layer_body(p_ref, dinv_ref, w1_ref, b1_ref, w2_ref, b2_ref, o_ref):
    i = pl.program_id(0)
    dinv = dinv_ref[0]
    t = (p_ref[0] + p_ref[1]) * dinv[:, None]
    z = jnp.dot(t, w1_ref[...], preferred_element_type=jnp.float32)
    z = jnp.maximum(z + b1_ref[...], 0.0)
    h = jnp.dot(z, w2_ref[...], preferred_element_type=jnp.float32) + b2_ref[...]
    o_ref[...] = jnp.where(_rowmask(i), h * dinv[:, None], 0.0)


def _tc_final_body(p_ref, dinv_ref, w1_ref, b1_ref, w2_ref, b2_ref,
                   g_ref, bb_ref, wo_ref, bo_ref, o_ref):
    dinv = dinv_ref[0]
    t = (p_ref[0] + p_ref[1]) * dinv[:, None]
    z = jnp.dot(t, w1_ref[...], preferred_element_type=jnp.float32)
    z = jnp.maximum(z + b1_ref[...], 0.0)
    h = jnp.dot(z, w2_ref[...], preferred_element_type=jnp.float32) + b2_ref[...]
    mu = jnp.mean(h, axis=-1, keepdims=True)
    var = jnp.mean((h - mu) ** 2, axis=-1, keepdims=True)
    hn = (h - mu) * lax.rsqrt(var + EPS) * g_ref[...] + bb_ref[...]
    o_ref[...] = jnp.dot(hn, wo_ref[...], preferred_element_type=jnp.float32) + bo_ref[...]


def _vec_spec():
    return pl.BlockSpec((1, D), lambda i: (0, 0))


def _mat_spec():
    return pl.BlockSpec((D, D), lambda i: (0, 0))


def _row_spec():
    return pl.BlockSpec((BR, D), lambda i: (i, 0))


def _dinv_spec():
    return pl.BlockSpec((1, BR), lambda i: (0, i))


def _part_spec():
    return pl.BlockSpec((NC, BR, D), lambda i: (0, i, 0))


_GRID = NP // BR

_tc_in = pl.pallas_call(
    _tc_in_body,
    grid=(_GRID,),
    in_specs=[_row_spec(), _mat_spec(), _vec_spec(),
              pl.BlockSpec((NW, BR), lambda i: (0, i))],
    out_specs=[_row_spec(), _dinv_spec()],
    out_shape=[jax.ShapeDtypeStruct((NP, D), jnp.float32),
               jax.ShapeDtypeStruct((1, NP), jnp.float32)],
)

_tc_layer = pl.pallas_call(
    _tc_layer_body,
    grid=(_GRID,),
    in_specs=[_part_spec(), _dinv_spec(), _mat_spec(), _vec_spec(),
              _mat_spec(), _vec_spec()],
    out_specs=_row_spec(),
    out_shape=jax.ShapeDtypeStruct((NP, D), jnp.float32),
)

_tc_final = pl.pallas_call(
    _tc_final_body,
    grid=(_GRID,),
    in_specs=[_part_spec(), _dinv_spec(), _mat_spec(), _vec_spec(),
              _mat_spec(), _vec_spec(), _vec_spec(), _vec_spec(),
              _mat_spec(), _vec_spec()],
    out_specs=_row_spec(),
    out_shape=jax.ShapeDtypeStruct((NP, D), jnp.float32),
)


def kernel(x, edge_index, W_in, b_in, W1_0, b1_0, W2_0, b2_0,
           W1_1, b1_1, W2_1, b2_1, ln_g, ln_b, W_out, b_out):
    src = edge_index[0]
    dst = edge_index[1]
    src_p = src.reshape(TOT_CHUNK, CH)
    dst_p = dst.reshape(TOT_CHUNK, CH)

    degp = _deg_call(dst)
    hs0, dinv = _tc_in(x, W_in, b_in.reshape(1, D), degp)
    p0 = _agg_call(hs0, src_p, dst_p)
    hs1 = _tc_layer(p0, dinv, W1_0, b1_0.reshape(1, D), W2_0, b2_0.reshape(1, D))
    p1 = _agg_call(hs1, src_p, dst_p)
    out = _tc_final(p1, dinv, W1_1, b1_1.reshape(1, D), W2_1, b2_1.reshape(1, D),
                    ln_g.reshape(1, D), ln_b.reshape(1, D), W_out, b_out.reshape(1, D))
    return out[:N]


# speedup vs baseline: 32.7689x; 1.0340x over previous
import functools

import jax
import jax.numpy as jnp
from jax import lax
from jax.experimental import pallas as pl
from jax.experimental.pallas import tpu as pltpu
from jax.experimental.pallas import tpu_sc as plsc

N = 10000
D = 128
NP = 10240
E = 320000
NC = 2
NS = 16
NW = NC * NS
CH = 128
TOT_CHUNK = E // CH
N0 = 1250
N1 = TOT_CHUNK - N0
EPT_DEG = E // NW
ACCR = 10048
FRT = 632
LRT = ACCR - 15 * FRT
BR = 2048
EPS = 1e-5

_mesh = plsc.VectorSubcoreMesh(core_axis_name="c", subcore_axis_name="s")
_sc_params = pltpu.CompilerParams(needs_layout_passes=False)


def _deg_body(dst_hbm, out_hbm, dstv, degv):
    c = lax.axis_index("c")
    s = lax.axis_index("s")
    wid = s * NC + c
    pltpu.sync_copy(dst_hbm.at[pl.ds(wid * EPT_DEG, EPT_DEG)], dstv)
    zeros16 = jnp.zeros((16,), jnp.float32)

    def zbody(i, carry):
        degv[pl.ds(i * 16, 16)] = zeros16
        return carry

    lax.fori_loop(0, NP // 16, zbody, 0)
    ones16 = jnp.ones((16,), jnp.float32)

    def ebody(j, carry):
        for k in range(5):
            idx = dstv[pl.ds(j * 80 + k * 16, 16)]
            plsc.addupdate_scatter(degv, [idx], ones16)
        return carry

    lax.fori_loop(0, EPT_DEG // 80, ebody, 0)
    pltpu.sync_copy(degv, out_hbm.at[wid])


_deg_call = functools.partial(
    pl.kernel,
    out_type=jax.ShapeDtypeStruct((NW, NP), jnp.float32),
    mesh=_mesh,
    compiler_params=_sc_params,
    scratch_types=[
        pltpu.VMEM((EPT_DEG,), jnp.int32),
        pltpu.VMEM((NP,), jnp.float32),
    ],
)(_deg_body)


def _range_of(c, s):
    k0, r0 = N0 // NS, N0 % NS
    k1, r1 = N1 // NS, N1 % NS
    base0 = s * k0 + jnp.minimum(s, r0)
    cnt0 = k0 + (s < r0).astype(jnp.int32)
    base1 = N0 + s * k1 + jnp.minimum(s, r1)
    cnt1 = k1 + (s < r1).astype(jnp.int32)
    base = jnp.where(c == 0, base0, base1)
    cnt = jnp.where(c == 0, cnt0, cnt1)
    return base, cnt


def _agg_body(hs_hbm, src_hbm, dst_hbm, out_hbm, sidx, didx, rows, acc_sh,
              isem, gsem, ssem):
    c = lax.axis_index("c")
    s = lax.axis_index("s")
    base, cnt = _range_of(c, s)
    zeros16 = jnp.zeros((16,), jnp.float32)

    def zbody(i, carry):
        for k in range(D // 16):
            rows[0, i, pl.ds(k * 16, 16)] = zeros16
        return carry

    lax.fori_loop(0, CH, zbody, 0)
    for q in range(4):
        pltpu.sync_copy(rows.at[0], acc_sh.at[pl.ds(s * FRT + q * CH, CH)])

    @pl.when(s < NS - 1)
    def _():
        pltpu.sync_copy(rows.at[0, pl.ds(0, FRT - 4 * CH)],
                        acc_sh.at[pl.ds(s * FRT + 4 * CH, FRT - 4 * CH)])

    @pl.when(s == NS - 1)
    def _():
        pltpu.sync_copy(rows.at[0, pl.ds(0, LRT - 4 * CH)],
                        acc_sh.at[pl.ds(s * FRT + 4 * CH, LRT - 4 * CH)])

    plsc.subcore_barrier()

    for p in range(3):
        pltpu.async_copy(src_hbm.at[base + p], sidx.at[p], isem.at[p])
        pltpu.async_copy(dst_hbm.at[base + p], didx.at[p], isem.at[p])
    pltpu.make_async_copy(src_hbm.at[base], sidx.at[0], isem.at[0]).wait()
    pltpu.make_async_copy(dst_hbm.at[base], didx.at[0], isem.at[0]).wait()
    pltpu.async_copy(hs_hbm.at[sidx.at[0]], rows.at[0], gsem.at[0])
    pltpu.make_async_copy(src_hbm.at[base + 1], sidx.at[1], isem.at[1]).wait()
    pltpu.make_async_copy(dst_hbm.at[base + 1], didx.at[1], isem.at[1]).wait()
    pltpu.async_copy(hs_hbm.at[sidx.at[1]], rows.at[1], gsem.at[1])

    def cbody(j, carry):
        b = lax.rem(j, 3)
        slot = lax.rem(j, 4)
        par = lax.rem(j, 2)
        pltpu.make_async_copy(hs_hbm.at[sidx.at[slot]], rows.at[b],
                              gsem.at[par]).wait()

        @pl.when(j >= 1)
        def _():
            pltpu.make_async_copy(rows.at[lax.rem(j + 2, 3)],
                                  acc_sh.at[didx.at[lax.rem(j + 3, 4)]],
                                  ssem).wait()

        @pl.when(j + 2 < cnt)
        def _():
            n2 = lax.rem(j + 2, 4)
            pltpu.make_async_copy(src_hbm.at[base + j + 2], sidx.at[n2],
                                  isem.at[n2]).wait()
            pltpu.make_async_copy(dst_hbm.at[base + j + 2], didx.at[n2],
                                  isem.at[n2]).wait()
            pltpu.async_copy(hs_hbm.at[sidx.at[n2]], rows.at[lax.rem(j + 2, 3)],
                             gsem.at[par])

        @pl.when(j + 3 < cnt)
        def _():
            n3 = lax.rem(j + 3, 4)
            pltpu.async_copy(src_hbm.at[base + j + 3], sidx.at[n3], isem.at[n3])
            pltpu.async_copy(dst_hbm.at[base + j + 3], didx.at[n3], isem.at[n3])

        pltpu.async_copy(rows.at[b], acc_sh.at[didx.at[slot]], ssem, add=True)
        return carry

    lax.fori_loop(0, cnt, cbody, 0)
    pltpu.make_async_copy(rows.at[lax.rem(cnt - 1, 3)],
                          acc_sh.at[didx.at[lax.rem(cnt - 1, 4)]], ssem).wait()
    plsc.subcore_barrier()

    @pl.when(s < NS - 1)
    def _():
        pltpu.sync_copy(acc_sh.at[pl.ds(s * FRT, FRT)],
                        out_hbm.at[c, pl.ds(s * FRT, FRT)])

    @pl.when(s == NS - 1)
    def _():
        pltpu.sync_copy(acc_sh.at[pl.ds(s * FRT, LRT)],
                        out_hbm.at[c, pl.ds(s * FRT, LRT)])


_agg_call = functools.partial(
    pl.kernel,
    out_type=jax.ShapeDtypeStruct((NC, ACCR, D), jnp.float32),
    mesh=_mesh,
    compiler_params=_sc_params,
    scratch_types=[
        pltpu.VMEM((4, CH), jnp.int32),
        pltpu.VMEM((4, CH), jnp.int32),
        pltpu.VMEM((3, CH, D), jnp.float32),
        pltpu.VMEM_SHARED((ACCR, D), jnp.float32),
        pltpu.SemaphoreType.DMA((4,)),
        pltpu.SemaphoreType.DMA((2,)),
        pltpu.SemaphoreType.DMA,
    ],
)(_agg_body)


def _rowmask(i):
    rid = lax.broadcasted_iota(jnp.int32, (BR, 1), 0) + i * BR
    return rid < N


def _tc_in_body(x_ref, w_ref, b_ref, degp_ref, o_ref, dinv_ref):
    i = pl.program_id(0)
    dinv = lax.rsqrt(jnp.maximum(jnp.sum(degp_ref[...], axis=0), 1.0))
    dinv_ref[...] = dinv[None, :]
    h = jnp.dot(x_ref[...], w_ref[...], preferred_element_type=jnp.float32)
    h = jnp.maximum(h + b_ref[...], 0.0)
    o_ref[...] = jnp.where(_rowmask(i), h * dinv[:, None], 0.0)


def _tc_layer_body(p_ref, dinv_ref, w1_ref, b1_ref, w2_ref, b2_ref, o_ref):
    i = pl.program_id(0)
    dinv = dinv_ref[0]
    t = (p_ref[0] + p_ref[1]) * dinv[:, None]
    z = jnp.dot(t, w1_ref[...], preferred_element_type=jnp.float32)
    z = jnp.maximum(z + b1_ref[...], 0.0)
    h = jnp.dot(z, w2_ref[...], preferred_element_type=jnp.float32) + b2_ref[...]
    o_ref[...] = jnp.where(_rowmask(i), h * dinv[:, None], 0.0)


def _tc_final_body(p_ref, dinv_ref, w1_ref, b1_ref, w2_ref, b2_ref,
                   g_ref, bb_ref, wo_ref, bo_ref, o_ref):
    dinv = dinv_ref[0]
    t = (p_ref[0] + p_ref[1]) * dinv[:, None]
    z = jnp.dot(t, w1_ref[...], preferred_element_type=jnp.float32)
    z = jnp.maximum(z + b1_ref[...], 0.0)
    h = jnp.dot(z, w2_ref[...], preferred_element_type=jnp.float32) + b2_ref[...]
    mu = jnp.mean(h, axis=-1, keepdims=True)
    var = jnp.mean((h - mu) ** 2, axis=-1, keepdims=True)
    hn = (h - mu) * lax.rsqrt(var + EPS) * g_ref[...] + bb_ref[...]
    o_ref[...] = jnp.dot(hn, wo_ref[...], preferred_element_type=jnp.float32) + bo_ref[...]


def _vec_spec():
    return pl.BlockSpec((1, D), lambda i: (0, 0))


def _mat_spec():
    return pl.BlockSpec((D, D), lambda i: (0, 0))


def _row_spec():
    return pl.BlockSpec((BR, D), lambda i: (i, 0))


def _dinv_spec():
    return pl.BlockSpec((1, BR), lambda i: (0, i))


def _part_spec():
    return pl.BlockSpec((NC, BR, D), lambda i: (0, i, 0))


_GRID = NP // BR

_tc_in = pl.pallas_call(
    _tc_in_body,
    grid=(_GRID,),
    in_specs=[_row_spec(), _mat_spec(), _vec_spec(),
              pl.BlockSpec((NW, BR), lambda i: (0, i))],
    out_specs=[_row_spec(), _dinv_spec()],
    out_shape=[jax.ShapeDtypeStruct((NP, D), jnp.float32),
               jax.ShapeDtypeStruct((1, NP), jnp.float32)],
)

_tc_layer = pl.pallas_call(
    _tc_layer_body,
    grid=(_GRID,),
    in_specs=[_part_spec(), _dinv_spec(), _mat_spec(), _vec_spec(),
              _mat_spec(), _vec_spec()],
    out_specs=_row_spec(),
    out_shape=jax.ShapeDtypeStruct((NP, D), jnp.float32),
)

_tc_final = pl.pallas_call(
    _tc_final_body,
    grid=(_GRID,),
    in_specs=[_part_spec(), _dinv_spec(), _mat_spec(), _vec_spec(),
              _mat_spec(), _vec_spec(), _vec_spec(), _vec_spec(),
              _mat_spec(), _vec_spec()],
    out_specs=_row_spec(),
    out_shape=jax.ShapeDtypeStruct((NP, D), jnp.float32),
)


def kernel(x, edge_index, W_in, b_in, W1_0, b1_0, W2_0, b2_0,
           W1_1, b1_1, W2_1, b2_1, ln_g, ln_b, W_out, b_out):
    src = edge_index[0]
    dst = edge_index[1]
    src_p = src.reshape(TOT_CHUNK, CH)
    dst_p = dst.reshape(TOT_CHUNK, CH)

    degp = _deg_call(dst)
    hs0, dinv = _tc_in(x, W_in, b_in.reshape(1, D), degp)
    p0 = _agg_call(hs0, src_p, dst_p)
    hs1 = _tc_layer(p0, dinv, W1_0, b1_0.reshape(1, D), W2_0, b2_0.reshape(1, D))
    p1 = _agg_call(hs1, src_p, dst_p)
    out = _tc_final(p1, dinv, W1_1, b1_1.reshape(1, D), W2_1, b2_1.reshape(1, D),
                    ln_g.reshape(1, D), ln_b.reshape(1, D), W_out, b_out.reshape(1, D))
    return out[:N]


# speedup vs baseline: 33.5855x; 1.0249x over previous
import functools

import jax
import jax.numpy as jnp
from jax import lax
from jax.experimental import pallas as pl
from jax.experimental.pallas import tpu as pltpu
from jax.experimental.pallas import tpu_sc as plsc

N = 10000
D = 128
NP = 10240
E = 320000
NC = 2
NS = 16
NW = NC * NS
CH = 128
TOT_CHUNK = E // CH
N0 = 1250
N1 = TOT_CHUNK - N0
EPT_DEG = E // NW
ACCR = 10048
FRT = 632
LRT = ACCR - 15 * FRT
BR = 5120
EPS = 1e-5

_mesh = plsc.VectorSubcoreMesh(core_axis_name="c", subcore_axis_name="s")
_sc_params = pltpu.CompilerParams(needs_layout_passes=False)


def _deg_body(dst_hbm, out_hbm, dstv, degv):
    c = lax.axis_index("c")
    s = lax.axis_index("s")
    wid = s * NC + c
    pltpu.sync_copy(dst_hbm.at[pl.ds(wid * EPT_DEG, EPT_DEG)], dstv)
    zeros16 = jnp.zeros((16,), jnp.float32)

    def zbody(i, carry):
        degv[pl.ds(i * 16, 16)] = zeros16
        return carry

    lax.fori_loop(0, NP // 16, zbody, 0)
    ones16 = jnp.ones((16,), jnp.float32)

    def ebody(j, carry):
        for k in range(5):
            idx = dstv[pl.ds(j * 80 + k * 16, 16)]
            plsc.addupdate_scatter(degv, [idx], ones16)
        return carry

    lax.fori_loop(0, EPT_DEG // 80, ebody, 0)
    pltpu.sync_copy(degv, out_hbm.at[wid])


_deg_call = functools.partial(
    pl.kernel,
    out_type=jax.ShapeDtypeStruct((NW, NP), jnp.float32),
    mesh=_mesh,
    compiler_params=_sc_params,
    scratch_types=[
        pltpu.VMEM((EPT_DEG,), jnp.int32),
        pltpu.VMEM((NP,), jnp.float32),
    ],
)(_deg_body)


def _range_of(c, s):
    k0, r0 = N0 // NS, N0 % NS
    k1, r1 = N1 // NS, N1 % NS
    base0 = s * k0 + jnp.minimum(s, r0)
    cnt0 = k0 + (s < r0).astype(jnp.int32)
    base1 = N0 + s * k1 + jnp.minimum(s, r1)
    cnt1 = k1 + (s < r1).astype(jnp.int32)
    base = jnp.where(c == 0, base0, base1)
    cnt = jnp.where(c == 0, cnt0, cnt1)
    return base, cnt


def _agg_body(hs_hbm, src_hbm, dst_hbm, out_hbm, sidx, didx, rows, acc_sh,
              isem, gsem, ssem):
    c = lax.axis_index("c")
    s = lax.axis_index("s")
    base, cnt = _range_of(c, s)
    zeros16 = jnp.zeros((16,), jnp.float32)

    def zbody(i, carry):
        for k in range(D // 16):
            rows[0, i, pl.ds(k * 16, 16)] = zeros16
        return carry

    lax.fori_loop(0, CH, zbody, 0)
    for q in range(4):
        pltpu.sync_copy(rows.at[0], acc_sh.at[pl.ds(s * FRT + q * CH, CH)])

    @pl.when(s < NS - 1)
    def _():
        pltpu.sync_copy(rows.at[0, pl.ds(0, FRT - 4 * CH)],
                        acc_sh.at[pl.ds(s * FRT + 4 * CH, FRT - 4 * CH)])

    @pl.when(s == NS - 1)
    def _():
        pltpu.sync_copy(rows.at[0, pl.ds(0, LRT - 4 * CH)],
                        acc_sh.at[pl.ds(s * FRT + 4 * CH, LRT - 4 * CH)])

    plsc.subcore_barrier()

    for p in range(3):
        pltpu.async_copy(src_hbm.at[base + p], sidx.at[p], isem.at[p])
        pltpu.async_copy(dst_hbm.at[base + p], didx.at[p], isem.at[p])
    pltpu.make_async_copy(src_hbm.at[base], sidx.at[0], isem.at[0]).wait()
    pltpu.make_async_copy(dst_hbm.at[base], didx.at[0], isem.at[0]).wait()
    pltpu.async_copy(hs_hbm.at[sidx.at[0]], rows.at[0], gsem.at[0])
    pltpu.make_async_copy(src_hbm.at[base + 1], sidx.at[1], isem.at[1]).wait()
    pltpu.make_async_copy(dst_hbm.at[base + 1], didx.at[1], isem.at[1]).wait()
    pltpu.async_copy(hs_hbm.at[sidx.at[1]], rows.at[1], gsem.at[1])

    def cbody(j, carry):
        b = lax.rem(j, 3)
        slot = lax.rem(j, 4)
        par = lax.rem(j, 2)
        pltpu.make_async_copy(hs_hbm.at[sidx.at[slot]], rows.at[b],
                              gsem.at[par]).wait()

        @pl.when(j >= 1)
        def _():
            pltpu.make_async_copy(rows.at[lax.rem(j + 2, 3)],
                                  acc_sh.at[didx.at[lax.rem(j + 3, 4)]],
                                  ssem).wait()

        @pl.when(j + 2 < cnt)
        def _():
            n2 = lax.rem(j + 2, 4)
            pltpu.make_async_copy(src_hbm.at[base + j + 2], sidx.at[n2],
                                  isem.at[n2]).wait()
            pltpu.make_async_copy(dst_hbm.at[base + j + 2], didx.at[n2],
                                  isem.at[n2]).wait()
            pltpu.async_copy(hs_hbm.at[sidx.at[n2]], rows.at[lax.rem(j + 2, 3)],
                             gsem.at[par])

        @pl.when(j + 3 < cnt)
        def _():
            n3 = lax.rem(j + 3, 4)
            pltpu.async_copy(src_hbm.at[base + j + 3], sidx.at[n3], isem.at[n3])
            pltpu.async_copy(dst_hbm.at[base + j + 3], didx.at[n3], isem.at[n3])

        pltpu.async_copy(rows.at[b], acc_sh.at[didx.at[slot]], ssem, add=True)
        return carry

    lax.fori_loop(0, cnt, cbody, 0)
    pltpu.make_async_copy(rows.at[lax.rem(cnt - 1, 3)],
                          acc_sh.at[didx.at[lax.rem(cnt - 1, 4)]], ssem).wait()
    plsc.subcore_barrier()

    @pl.when(s < NS - 1)
    def _():
        pltpu.sync_copy(acc_sh.at[pl.ds(s * FRT, FRT)],
                        out_hbm.at[c, pl.ds(s * FRT, FRT)])

    @pl.when(s == NS - 1)
    def _():
        pltpu.sync_copy(acc_sh.at[pl.ds(s * FRT, LRT)],
                        out_hbm.at[c, pl.ds(s * FRT, LRT)])


_agg_call = functools.partial(
    pl.kernel,
    out_type=jax.ShapeDtypeStruct((NC, ACCR, D), jnp.float32),
    mesh=_mesh,
    compiler_params=_sc_params,
    scratch_types=[
        pltpu.VMEM((4, CH), jnp.int32),
        pltpu.VMEM((4, CH), jnp.int32),
        pltpu.VMEM((3, CH, D), jnp.float32),
        pltpu.VMEM_SHARED((ACCR, D), jnp.float32),
        pltpu.SemaphoreType.DMA((4,)),
        pltpu.SemaphoreType.DMA((2,)),
        pltpu.SemaphoreType.DMA,
    ],
)(_agg_body)


def _rowmask(i):
    rid = lax.broadcasted_iota(jnp.int32, (BR, 1), 0) + i * BR
    return rid < N


def _tc_in_body(x_ref, w_ref, b_ref, degp_ref, o_ref, dinv_ref):
    i = pl.program_id(0)
    dinv = lax.rsqrt(jnp.maximum(jnp.sum(degp_ref[...], axis=0), 1.0))
    dinv_ref[...] = dinv[None, :]
    h = jnp.dot(x_ref[...], w_ref[...], preferred_element_type=jnp.float32)
    h = jnp.maximum(h + b_ref[...], 0.0)
    o_ref[...] = jnp.where(_rowmask(i), h * dinv[:, None], 0.0)


def _tc_layer_body(p_ref, dinv_ref, w1_ref, b1_ref, w2_ref, b2_ref, o_ref):
    i = pl.program_id(0)
    dinv = dinv_ref[0]
    t = (p_ref[0] + p_ref[1]) * dinv[:, None]
    z = jnp.dot(t, w1_ref[...], preferred_element_type=jnp.float32)
    z = jnp.maximum(z + b1_ref[...], 0.0)
    h = jnp.dot(z, w2_ref[...], preferred_element_type=jnp.float32) + b2_ref[...]
    o_ref[...] = jnp.where(_rowmask(i), h * dinv[:, None], 0.0)


def _tc_final_body(p_ref, dinv_ref, w1_ref, b1_ref, w2_ref, b2_ref,
                   g_ref, bb_ref, wo_ref, bo_ref, o_ref):
    dinv = dinv_ref[0]
    t = (p_ref[0] + p_ref[1]) * dinv[:, None]
    z = jnp.dot(t, w1_ref[...], preferred_element_type=jnp.float32)
    z = jnp.maximum(z + b1_ref[...], 0.0)
    h = jnp.dot(z, w2_ref[...], preferred_element_type=jnp.float32) + b2_ref[...]
    mu = jnp.mean(h, axis=-1, keepdims=True)
    var = jnp.mean((h - mu) ** 2, axis=-1, keepdims=True)
    hn = (h - mu) * lax.rsqrt(var + EPS) * g_ref[...] + bb_ref[...]
    o_ref[...] = jnp.dot(hn, wo_ref[...], preferred_element_type=jnp.float32) + bo_ref[...]


def _vec_spec():
    return pl.BlockSpec((1, D), lambda i: (0, 0))


def _mat_spec():
    return pl.BlockSpec((D, D), lambda i: (0, 0))


def _row_spec():
    return pl.BlockSpec((BR, D), lambda i: (i, 0))


def _dinv_spec():
    return pl.BlockSpec((1, BR), lambda i: (0, i))


def _part_spec():
    return pl.BlockSpec((NC, BR, D), lambda i: (0, i, 0))


_GRID = NP // BR

_tc_in = pl.pallas_call(
    _tc_in_body,
    grid=(_GRID,),
    in_specs=[_row_spec(), _mat_spec(), _vec_spec(),
              pl.BlockSpec((NW, BR), lambda i: (0, i))],
    out_specs=[_row_spec(), _dinv_spec()],
    out_shape=[jax.ShapeDtypeStruct((NP, D), jnp.float32),
               jax.ShapeDtypeStruct((1, NP), jnp.float32)],
)

_tc_layer = pl.pallas_call(
    _tc_layer_body,
    grid=(_GRID,),
    in_specs=[_part_spec(), _dinv_spec(), _mat_spec(), _vec_spec(),
              _mat_spec(), _vec_spec()],
    out_specs=_row_spec(),
    out_shape=jax.ShapeDtypeStruct((NP, D), jnp.float32),
)

_tc_final = pl.pallas_call(
    _tc_final_body,
    grid=(_GRID,),
    in_specs=[_part_spec(), _dinv_spec(), _mat_spec(), _vec_spec(),
              _mat_spec(), _vec_spec(), _vec_spec(), _vec_spec(),
              _mat_spec(), _vec_spec()],
    out_specs=_row_spec(),
    out_shape=jax.ShapeDtypeStruct((NP, D), jnp.float32),
)


def kernel(x, edge_index, W_in, b_in, W1_0, b1_0, W2_0, b2_0,
           W1_1, b1_1, W2_1, b2_1, ln_g, ln_b, W_out, b_out):
    src = edge_index[0]
    dst = edge_index[1]
    src_p = src.reshape(TOT_CHUNK, CH)
    dst_p = dst.reshape(TOT_CHUNK, CH)

    degp = _deg_call(dst)
    hs0, dinv = _tc_in(x, W_in, b_in.reshape(1, D), degp)
    p0 = _agg_call(hs0, src_p, dst_p)
    hs1 = _tc_layer(p0, dinv, W1_0, b1_0.reshape(1, D), W2_0, b2_0.reshape(1, D))
    p1 = _agg_call(hs1, src_p, dst_p)
    out = _tc_final(p1, dinv, W1_1, b1_1.reshape(1, D), W2_1, b2_1.reshape(1, D),
                    ln_g.reshape(1, D), ln_b.reshape(1, D), W_out, b_out.reshape(1, D))
    return out[:N]
